# baseline (speedup 1.0000x reference)
import functools

import jax
import jax.numpy as jnp
from jax import lax
from jax.experimental import pallas as pl
from jax.experimental.pallas import tpu as pltpu
from jax.experimental.pallas import tpu_sc as plsc

_H = 128
_I = 128
_E = 8
_E_LOCAL = 2
_BT = 4096


def _routing_combine_sc(rl_t, T):
    info = plsc.get_sparse_core_info()
    NC, NS, L = info.num_cores, info.num_subcores, info.num_lanes
    NW = NC * NS
    TW = T // NW
    mesh = plsc.VectorSubcoreMesh(core_axis_name="c", subcore_axis_name="s")

    @functools.partial(
        pl.kernel, mesh=mesh,
        out_type=jax.ShapeDtypeStruct((2, T), jnp.float32),
        scratch_types=[
            pltpu.VMEM((_E, TW), jnp.float32),
            pltpu.VMEM((2, TW), jnp.float32),
        ],
    )
    def k(rl_hbm, out_hbm, rl_v, out_v):
        wid = lax.axis_index("s") * NC + lax.axis_index("c")
        base = wid * TW
        pltpu.sync_copy(rl_hbm.at[:, pl.ds(base, TW)], rl_v)

        def body(i, carry):
            vs = [rl_v[e, pl.ds(i * L, L)] for e in range(_E)]
            m1 = vs[0]
            for e in range(1, _E):
                m1 = jnp.maximum(m1, vs[e])
            neg_inf = jnp.float32(-jnp.inf)
            cnt = jnp.zeros((L,), jnp.float32)
            m2 = jnp.full((L,), neg_inf, jnp.float32)
            for e in range(_E):
                is1 = vs[e] == m1
                cnt = cnt + jnp.where(is1, 1.0, 0.0)
                m2 = jnp.maximum(m2, jnp.where(is1, neg_inf, vs[e]))
            m2 = jnp.where(cnt >= 2.0, m1, m2)
            e2 = jnp.exp(m2 - m1)
            wa = 1.0 / (1.0 + e2)
            wb = e2 * wa
            f1_0 = jnp.where(vs[0] == m1, 1.0, 0.0)
            f1_1 = jnp.where(vs[1] == m1, 1.0, 0.0) * (1.0 - f1_0)
            f2_0 = jnp.where(vs[0] == m2, 1.0, 0.0) * (1.0 - f1_0)
            f2_1 = (jnp.where(vs[1] == m2, 1.0, 0.0) * (1.0 - f1_1)
                    * (1.0 - f2_0))
            c0 = f1_0 * wa + f2_0 * wb
            c1 = f1_1 * wa + f2_1 * wb
            out_v[0, pl.ds(i * L, L)] = c0
            out_v[1, pl.ds(i * L, L)] = c1
            return carry

        lax.fori_loop(0, TW // L, body, 0)
        pltpu.sync_copy(out_v, out_hbm.at[:, pl.ds(base, TW)])

    return k(rl_t)


def _moe_block_kernel(x_ref, comb_ref, w13c_ref, w13bc_ref, w2c_ref, w2b_ref,
                      y_ref):
    x = x_ref[...]
    c0 = comb_ref[:, 0:1]
    c1 = comb_ref[:, 1:2]
    xb = x.astype(jnp.bfloat16)
    gu = (jnp.dot(xb, w13c_ref[...], preferred_element_type=jnp.float32)
          + w13bc_ref[...])
    acts = []
    ybias = jnp.zeros((x.shape[0], _H), dtype=jnp.float32)
    for e, comb in ((0, c0), (1, c1)):
        g = gu[:, 2 * _I * e: 2 * _I * e + _I]
        u = gu[:, 2 * _I * e + _I: 2 * _I * (e + 1)]
        act = g * jax.nn.sigmoid(g) * u
        acts.append((comb * act).astype(jnp.bfloat16))
        ybias = ybias + comb * w2b_ref[e:e + 1, :]
    actcat = jnp.concatenate(acts, axis=-1)
    y_ref[...] = (jnp.dot(actcat, w2c_ref[...],
                          preferred_element_type=jnp.float32) + ybias)


def kernel(x, router_logits, w13, w13_bias, w2, w2_bias):
    T = x.shape[0]
    rl_t = router_logits.T
    w13c = jnp.transpose(w13, (2, 0, 1)).reshape(_H, _E_LOCAL * 2 * _I)
    w13c = w13c.astype(jnp.bfloat16)
    w13bc = w13_bias.reshape(1, _E_LOCAL * 2 * _I)
    w2c = jnp.transpose(w2, (0, 2, 1)).reshape(_E_LOCAL * _I, _H)
    w2c = w2c.astype(jnp.bfloat16)

    def tc_half(xh, combh):
        return pl.pallas_call(
            _moe_block_kernel,
            grid=(xh.shape[0] // _BT,),
            in_specs=[
                pl.BlockSpec((_BT, _H), lambda i: (i, 0)),
                pl.BlockSpec((_BT, 2), lambda i: (i, 0)),
                pl.BlockSpec((_H, _E_LOCAL * 2 * _I), lambda i: (0, 0)),
                pl.BlockSpec((1, _E_LOCAL * 2 * _I), lambda i: (0, 0)),
                pl.BlockSpec((_E_LOCAL * _I, _H), lambda i: (0, 0)),
                pl.BlockSpec((_E_LOCAL, _H), lambda i: (0, 0)),
            ],
            out_specs=pl.BlockSpec((_BT, _H), lambda i: (i, 0)),
            out_shape=jax.ShapeDtypeStruct((xh.shape[0], _H), jnp.float32),
        )(xh, combh, w13c, w13bc, w2c, w2_bias)

    half = T // 2
    comb_a = _routing_combine_sc(rl_t[:, :half], half).T
    comb_b = _routing_combine_sc(rl_t[:, half:], half).T
    y_a = tc_half(x[:half], comb_a)
    y_b = tc_half(x[half:], comb_b)
    return jnp.concatenate([y_a, y_b], axis=0)


# speedup vs baseline: 1.5263x; 1.5263x over previous
import functools

import jax
import jax.numpy as jnp
from jax import lax
from jax.experimental import pallas as pl
from jax.experimental.pallas import tpu as pltpu
from jax.experimental.pallas import tpu_sc as plsc

_H = 128
_I = 128
_E = 8
_E_LOCAL = 2
_BT = 8192


def _routing_combine_sc(rl_t, T):
    info = plsc.get_sparse_core_info()
    NC, NS, L = info.num_cores, info.num_subcores, info.num_lanes
    NW = NC * NS
    TW = T // NW
    mesh = plsc.VectorSubcoreMesh(core_axis_name="c", subcore_axis_name="s")

    @functools.partial(
        pl.kernel, mesh=mesh,
        out_type=jax.ShapeDtypeStruct((2, T), jnp.float32),
        scratch_types=[
            pltpu.VMEM((_E, TW), jnp.float32),
            pltpu.VMEM((2, TW), jnp.float32),
        ],
    )
    def k(rl_hbm, out_hbm, rl_v, out_v):
        wid = lax.axis_index("s") * NC + lax.axis_index("c")
        base = wid * TW
        pltpu.sync_copy(rl_hbm.at[:, pl.ds(base, TW)], rl_v)

        def body(i, carry):
            vs = [rl_v[e, pl.ds(i * L, L)] for e in range(_E)]
            m1 = vs[0]
            for e in range(1, _E):
                m1 = jnp.maximum(m1, vs[e])
            neg_inf = jnp.float32(-jnp.inf)
            cnt = jnp.zeros((L,), jnp.float32)
            m2 = jnp.full((L,), neg_inf, jnp.float32)
            for e in range(_E):
                is1 = vs[e] == m1
                cnt = cnt + jnp.where(is1, 1.0, 0.0)
                m2 = jnp.maximum(m2, jnp.where(is1, neg_inf, vs[e]))
            m2 = jnp.where(cnt >= 2.0, m1, m2)
            e2 = jnp.exp(m2 - m1)
            wa = 1.0 / (1.0 + e2)
            wb = e2 * wa
            f1_0 = jnp.where(vs[0] == m1, 1.0, 0.0)
            f1_1 = jnp.where(vs[1] == m1, 1.0, 0.0) * (1.0 - f1_0)
            f2_0 = jnp.where(vs[0] == m2, 1.0, 0.0) * (1.0 - f1_0)
            f2_1 = (jnp.where(vs[1] == m2, 1.0, 0.0) * (1.0 - f1_1)
                    * (1.0 - f2_0))
            c0 = f1_0 * wa + f2_0 * wb
            c1 = f1_1 * wa + f2_1 * wb
            out_v[0, pl.ds(i * L, L)] = c0
            out_v[1, pl.ds(i * L, L)] = c1
            return carry

        lax.fori_loop(0, TW // L, body, 0)
        pltpu.sync_copy(out_v, out_hbm.at[:, pl.ds(base, TW)])

    return k(rl_t)


def _moe_block_kernel(x_ref, comb_ref, w13c_ref, w13bc_ref, w2c_ref, w2b_ref,
                      y_ref):
    x = x_ref[...]
    c0 = comb_ref[:, 0:1]
    c1 = comb_ref[:, 1:2]
    xb = x.astype(jnp.bfloat16)
    gu = (jnp.dot(xb, w13c_ref[...], preferred_element_type=jnp.float32)
          + w13bc_ref[...])
    acts = []
    ybias = jnp.zeros((x.shape[0], _H), dtype=jnp.float32)
    for e, comb in ((0, c0), (1, c1)):
        g = gu[:, 2 * _I * e: 2 * _I * e + _I]
        u = gu[:, 2 * _I * e + _I: 2 * _I * (e + 1)]
        act = g * jax.nn.sigmoid(g) * u
        acts.append((comb * act).astype(jnp.bfloat16))
        ybias = ybias + comb * w2b_ref[e:e + 1, :]
    actcat = jnp.concatenate(acts, axis=-1)
    y_ref[...] = (jnp.dot(actcat, w2c_ref[...],
                          preferred_element_type=jnp.float32) + ybias)


def kernel(x, router_logits, w13, w13_bias, w2, w2_bias):
    T = x.shape[0]
    comb = _routing_combine_sc(router_logits.T, T).T
    w13c = jnp.transpose(w13, (2, 0, 1)).reshape(_H, _E_LOCAL * 2 * _I)
    w13c = w13c.astype(jnp.bfloat16)
    w13bc = w13_bias.reshape(1, _E_LOCAL * 2 * _I)
    w2c = jnp.transpose(w2, (0, 2, 1)).reshape(_E_LOCAL * _I, _H)
    w2c = w2c.astype(jnp.bfloat16)
    return pl.pallas_call(
        _moe_block_kernel,
        grid=(T // _BT,),
        in_specs=[
            pl.BlockSpec((_BT, _H), lambda i: (i, 0)),
            pl.BlockSpec((_BT, 2), lambda i: (i, 0)),
            pl.BlockSpec((_H, _E_LOCAL * 2 * _I), lambda i: (0, 0)),
            pl.BlockSpec((1, _E_LOCAL * 2 * _I), lambda i: (0, 0)),
            pl.BlockSpec((_E_LOCAL * _I, _H), lambda i: (0, 0)),
            pl.BlockSpec((_E_LOCAL, _H), lambda i: (0, 0)),
        ],
        out_specs=pl.BlockSpec((_BT, _H), lambda i: (i, 0)),
        out_shape=jax.ShapeDtypeStruct((T, _H), jnp.float32),
    )(x, comb, w13c, w13bc, w2c, w2_bias)
